# Initial kernel scaffold; baseline (speedup 1.0000x reference)
import functools

import jax
import jax.numpy as jnp
from jax import lax
from jax.experimental import pallas as pl
from jax.experimental.pallas import tpu as pltpu
from jax.experimental.pallas import tpu_sc as plsc

R = 128
N = 32768
K = 3276
L = 16
NV = N // L
NW = 32
ROWS_PER_W = R // NW

_SQRT_MAGIC = jnp.int32(0x1FBD1DF5)


def _sc_body(x_hbm, out_hbm, row_v, cand_v, cand2_v, hist_v):
    c = lax.axis_index("c")
    s = lax.axis_index("s")
    wid = s * 2 + c
    lanes = lax.iota(jnp.int32, L)
    ones_i = jnp.ones((L,), jnp.int32)

    def zero_hist(i, carry):
        hist_v[pl.ds(i * L, L)] = jnp.zeros((L,), jnp.int32)
        return carry

    def scan_hist(t, carry):
        cum, dsel, above, found, rank = carry
        d = jnp.int32(255) - t
        cd = jnp.sum(hist_v[pl.ds(d * L, L)])
        newcum = cum + cd
        cross = jnp.logical_and(jnp.logical_not(found), newcum >= rank)
        dsel = jnp.where(cross, d, dsel)
        above = jnp.where(cross, cum, above)
        found = jnp.logical_or(found, cross)
        return (newcum, dsel, above, found, rank)

    def do_row(j, carry):
        row = wid * ROWS_PER_W + j
        pltpu.sync_copy(x_hbm.at[row], row_v)

        lax.fori_loop(0, 256, zero_hist, 0)

        def p1(i, carry):
            v = row_v[pl.ds(i * L, L)]
            b = plsc.bitcast(jnp.abs(v), jnp.int32)
            d = lax.shift_right_logical(b, 23)
            idx = (d << 4) | lanes
            plsc.addupdate_scatter(hist_v, [idx], ones_i)
            return carry

        lax.fori_loop(0, NV, p1, 0)
        _, d1, a1, _, _ = lax.fori_loop(
            0, 256, scan_hist,
            (jnp.int32(0), jnp.int32(0), jnp.int32(0), False, jnp.int32(K)))
        r1 = jnp.int32(K) - a1

        hi = (d1 + jnp.int32(1)) << 23

        def cpc(i, carry):
            off, ss = carry
            v = row_v[pl.ds(i * L, L)]
            a = jnp.abs(v)
            b = plsc.bitcast(a, jnp.int32)
            ss = ss + jnp.where(b >= hi, a * a, jnp.float32(0.0))
            m = lax.shift_right_logical(b, 23) == d1
            plsc.store_compressed(cand_v.at[pl.ds(off, L)], a, m)
            cnt = jnp.max(plsc.all_reduce_population_count(m))
            return (off + cnt, ss)

        n_cand, ss_vec = lax.fori_loop(
            0, NV, cpc, (jnp.int32(0), jnp.zeros((L,), jnp.float32)))
        nv2 = lax.div(n_cand + jnp.int32(L - 1), jnp.int32(L))

        lax.fori_loop(0, 256, zero_hist, 0)

        def p2(i, carry):
            a = cand_v[pl.ds(i * L, L)]
            b = plsc.bitcast(a, jnp.int32)
            d = lax.shift_right_logical(b, 15) & jnp.int32(255)
            idx = (d << 4) | lanes
            tm = (i * L + lanes) < n_cand
            plsc.addupdate_scatter(hist_v, [idx], ones_i, mask=tm)
            return carry

        lax.fori_loop(0, nv2, p2, 0)
        _, d2, a2, _, _ = lax.fori_loop(
            0, 256, scan_hist,
            (jnp.int32(0), jnp.int32(0), jnp.int32(0), False, r1))
        r2 = r1 - a2

        def cpc2(i, off2):
            a = cand_v[pl.ds(i * L, L)]
            b = plsc.bitcast(a, jnp.int32)
            tm = (i * L + lanes) < n_cand
            m = jnp.logical_and(
                tm, (lax.shift_right_logical(b, 15) & jnp.int32(255)) == d2)
            plsc.store_compressed(cand2_v.at[pl.ds(off2, L)], a, m)
            return off2 + jnp.max(plsc.all_reduce_population_count(m))

        n2 = lax.fori_loop(0, nv2, cpc2, jnp.int32(0))
        nv3 = lax.div(n2 + jnp.int32(L - 1), jnp.int32(L))

        base = (d1 << 23) | (d2 << 15)

        def bitstep(t, acc):
            bit = jnp.int32(1) << (jnp.int32(14) - t)
            trial = base | acc | bit

            def cnt_body(i, cv):
                a = cand2_v[pl.ds(i * L, L)]
                b = plsc.bitcast(a, jnp.int32)
                tm = (i * L + lanes) < n2
                keep = jnp.logical_and(tm, b >= trial)
                return cv + jnp.where(keep, jnp.int32(1), jnp.int32(0))

            cv = lax.fori_loop(0, nv3, cnt_body, jnp.zeros((L,), jnp.int32))
            return jnp.where(jnp.sum(cv) >= r2, acc | bit, acc)

        tbits = base | lax.fori_loop(0, 15, bitstep, jnp.int32(0))

        def ssc(i, ss):
            a = cand_v[pl.ds(i * L, L)]
            b = plsc.bitcast(a, jnp.int32)
            tm = (i * L + lanes) < n_cand
            m = jnp.logical_and(tm, b >= tbits)
            return ss + jnp.where(m, a * a, jnp.float32(0.0))

        ss_vec = lax.fori_loop(0, nv2, ssc, ss_vec)
        ss = jnp.sum(ss_vec)

        s_vec = jnp.full((L,), ss, jnp.float32)
        y = plsc.bitcast(
            lax.shift_right_logical(plsc.bitcast(s_vec, jnp.int32), 1)
            + _SQRT_MAGIC, jnp.float32)
        for _ in range(4):
            y = jnp.float32(0.5) * (y + s_vec / y)
        inv = jnp.float32(1.0) / (y + jnp.float32(1e-6))

        def sc(i, carry):
            v = row_v[pl.ds(i * L, L)]
            b = plsc.bitcast(jnp.abs(v), jnp.int32)
            row_v[pl.ds(i * L, L)] = jnp.where(
                b >= tbits, v * inv, jnp.float32(0.0))
            return carry

        lax.fori_loop(0, NV, sc, 0)
        pltpu.sync_copy(row_v, out_hbm.at[row])
        return carry

    lax.fori_loop(0, ROWS_PER_W, do_row, 0)


_sc_topk = functools.partial(
    pl.kernel,
    out_type=jax.ShapeDtypeStruct((R, N), jnp.float32),
    mesh=plsc.VectorSubcoreMesh(core_axis_name="c", subcore_axis_name="s"),
    scratch_types=[
        pltpu.VMEM((N,), jnp.float32),
        pltpu.VMEM((N,), jnp.float32),
        pltpu.VMEM((N,), jnp.float32),
        pltpu.VMEM((256 * L,), jnp.int32),
    ],
)(_sc_body)


def kernel(x):
    return _sc_topk(x)


# speedup vs baseline: 5.2583x; 5.2583x over previous
import functools

import jax
import jax.numpy as jnp
from jax import lax
from jax.experimental import pallas as pl
from jax.experimental.pallas import tpu as pltpu
from jax.experimental.pallas import tpu_sc as plsc

R = 128
N = 32768
K = 3276
L = 16
NV = N // L
NW = 32
ROWS_PER_W = R // NW

_SQRT_MAGIC = 0x1FBD1DF5


def _sc_body(x_hbm, out_hbm, row_v, cand_v, cand2_v, hist_v, counts_s):
    c = lax.axis_index("c")
    s = lax.axis_index("s")
    wid = s * 2 + c
    lanes = lax.iota(jnp.int32, L)
    ones_i = jnp.ones((L,), jnp.int32)

    def zero_hist(i, carry):
        hist_v[pl.ds(i * L, L)] = jnp.zeros((L,), jnp.int32)
        return carry

    def lane_reduce(d, carry):
        counts_s[d] = jnp.sum(hist_v[pl.ds(d * L, L)])
        return carry

    def scan_hist(t, carry):
        cum, dsel, above, found, rank = carry
        d = jnp.int32(255) - t
        cd = counts_s[d]
        newcum = cum + cd
        cross = jnp.logical_and(jnp.logical_not(found), newcum >= rank)
        dsel = jnp.where(cross, d, dsel)
        above = jnp.where(cross, cum, above)
        found = jnp.logical_or(found, cross)
        return (newcum, dsel, above, found, rank)

    def find_digit(rank):
        lax.fori_loop(0, 256, lane_reduce, 0)
        _, dsel, above, _, _ = lax.fori_loop(
            0, 256, scan_hist,
            (jnp.int32(0), jnp.int32(0), jnp.int32(0), False, rank))
        return dsel, above

    def do_row(j, carry):
        row = wid * ROWS_PER_W + j
        pltpu.sync_copy(x_hbm.at[row], row_v)

        lax.fori_loop(0, 256, zero_hist, 0)

        def p1(i, carry):
            v = row_v[pl.ds(i * L, L)]
            b = plsc.bitcast(jnp.abs(v), jnp.int32)
            d = lax.shift_right_logical(b, 23)
            idx = (d << 4) | lanes
            plsc.addupdate_scatter(hist_v, [idx], ones_i)
            return carry

        lax.fori_loop(0, NV, p1, 0)
        d1, a1 = find_digit(jnp.int32(K))
        r1 = jnp.int32(K) - a1

        hi = (d1 + jnp.int32(1)) << 23

        def cpc(i, carry):
            off, ss = carry
            v = row_v[pl.ds(i * L, L)]
            a = jnp.abs(v)
            b = plsc.bitcast(a, jnp.int32)
            ss = ss + jnp.where(b >= hi, a * a, jnp.float32(0.0))
            m = lax.shift_right_logical(b, 23) == d1
            pos = plsc.cumsum(jnp.where(m, jnp.int32(1), jnp.int32(0)))
            plsc.store_scatter(cand_v, [off + pos - 1], a, mask=m)
            return (off + plsc.all_reduce_population_count(m), ss)

        n_cand_vec, ss_vec = lax.fori_loop(
            0, NV, cpc,
            (jnp.zeros((L,), jnp.int32), jnp.zeros((L,), jnp.float32)))
        n_cand = jnp.max(n_cand_vec)
        nv2 = lax.div(n_cand + jnp.int32(L - 1), jnp.int32(L))

        lax.fori_loop(0, 256, zero_hist, 0)

        def p2(i, carry):
            a = cand_v[pl.ds(i * L, L)]
            b = plsc.bitcast(a, jnp.int32)
            d = lax.shift_right_logical(b, 15) & jnp.int32(255)
            idx = (d << 4) | lanes
            tm = (i * L + lanes) < n_cand
            plsc.addupdate_scatter(hist_v, [idx], ones_i, mask=tm)
            return carry

        lax.fori_loop(0, nv2, p2, 0)
        d2, a2 = find_digit(r1)
        r2 = r1 - a2

        hi2 = ((d1 << 23) | (d2 << 15)) + jnp.int32(1 << 15)

        def cpc2(i, carry):
            off2, ss = carry
            a = cand_v[pl.ds(i * L, L)]
            b = plsc.bitcast(a, jnp.int32)
            tm = (i * L + lanes) < n_cand
            ss = ss + jnp.where(jnp.logical_and(tm, b >= hi2), a * a,
                                jnp.float32(0.0))
            m = jnp.logical_and(
                tm, (lax.shift_right_logical(b, 15) & jnp.int32(255)) == d2)
            pos = plsc.cumsum(jnp.where(m, jnp.int32(1), jnp.int32(0)))
            plsc.store_scatter(cand2_v, [off2 + pos - 1], a, mask=m)
            return (off2 + plsc.all_reduce_population_count(m), ss)

        n2_vec, ss_vec = lax.fori_loop(
            0, nv2, cpc2, (jnp.zeros((L,), jnp.int32), ss_vec))
        n2 = jnp.max(n2_vec)
        nv3 = lax.div(n2 + jnp.int32(L - 1), jnp.int32(L))

        base = (d1 << 23) | (d2 << 15)

        def bitstep(t, acc):
            bit = jnp.int32(1) << (jnp.int32(14) - t)
            trial = base | acc | bit

            def cnt_body(i, cv):
                a = cand2_v[pl.ds(i * L, L)]
                b = plsc.bitcast(a, jnp.int32)
                tm = (i * L + lanes) < n2
                keep = jnp.logical_and(tm, b >= trial)
                return cv + jnp.where(keep, jnp.int32(1), jnp.int32(0))

            cv = lax.fori_loop(0, nv3, cnt_body, jnp.zeros((L,), jnp.int32))
            return jnp.where(jnp.sum(cv) >= r2, acc | bit, acc)

        tbits = base | lax.fori_loop(0, 15, bitstep, jnp.int32(0))

        def ssc(i, ss):
            a = cand2_v[pl.ds(i * L, L)]
            b = plsc.bitcast(a, jnp.int32)
            tm = (i * L + lanes) < n2
            m = jnp.logical_and(tm, b >= tbits)
            return ss + jnp.where(m, a * a, jnp.float32(0.0))

        ss_vec = lax.fori_loop(0, nv3, ssc, ss_vec)
        ss = jnp.sum(ss_vec)

        s_vec = jnp.full((L,), ss, jnp.float32)
        y = plsc.bitcast(
            lax.shift_right_logical(plsc.bitcast(s_vec, jnp.int32), 1)
            + _SQRT_MAGIC, jnp.float32)
        for _ in range(4):
            y = jnp.float32(0.5) * (y + s_vec / y)
        inv = jnp.float32(1.0) / (y + jnp.float32(1e-6))

        def sc(i, carry):
            v = row_v[pl.ds(i * L, L)]
            b = plsc.bitcast(jnp.abs(v), jnp.int32)
            row_v[pl.ds(i * L, L)] = jnp.where(
                b >= tbits, v * inv, jnp.float32(0.0))
            return carry

        lax.fori_loop(0, NV, sc, 0)
        pltpu.sync_copy(row_v, out_hbm.at[row])
        return carry

    lax.fori_loop(0, ROWS_PER_W, do_row, 0)


_sc_topk = functools.partial(
    pl.kernel,
    out_type=jax.ShapeDtypeStruct((R, N), jnp.float32),
    mesh=plsc.VectorSubcoreMesh(core_axis_name="c", subcore_axis_name="s"),
    scratch_types=[
        pltpu.VMEM((N,), jnp.float32),
        pltpu.VMEM((N,), jnp.float32),
        pltpu.VMEM((N,), jnp.float32),
        pltpu.VMEM((256 * L,), jnp.int32),
        pltpu.SMEM((256,), jnp.int32),
    ],
    compiler_params=pltpu.CompilerParams(needs_layout_passes=False),
)(_sc_body)


def kernel(x):
    return _sc_topk(x)
